# baseline (speedup 1.0000x reference)
import functools

import jax
import jax.numpy as jnp
from jax import lax
from jax.experimental import pallas as pl
from jax.experimental.pallas import tpu as pltpu
from jax.experimental.pallas import tpu_sc as plsc

B = 1024
P = 5
N = 10
DIM = 32
NPOS = P + N
TOT = (NPOS + 1) * B

_NC = 2
_NS = 16
_NW = _NC * _NS
ROWS_PER_W = TOT // _NW
CHUNK = 128
NCHUNK = ROWS_PER_W // CHUNK


@functools.cache
def _make_sc_gather():
    @functools.partial(
        pl.kernel,
        out_type=jax.ShapeDtypeStruct((TOT, DIM), jnp.float32),
        mesh=plsc.VectorSubcoreMesh(core_axis_name="c", subcore_axis_name="s"),
        scratch_types=[
            pltpu.VMEM((NCHUNK, CHUNK), jnp.int32),
            pltpu.VMEM((ROWS_PER_W, DIM), jnp.float32),
            pltpu.SemaphoreType.DMA,
        ],
        compiler_params=pltpu.CompilerParams(use_tc_tiling_on_sc=False),
    )
    def _sc_gather(idx_hbm, table_hbm, out_hbm, idx_v, rows_v, sem):
        wid = lax.axis_index("s") * _NC + lax.axis_index("c")
        base = wid * ROWS_PER_W
        pltpu.sync_copy(idx_hbm.at[pl.ds(wid * NCHUNK, NCHUNK)], idx_v)
        copies = []
        for j in range(NCHUNK):
            copies.append(
                pltpu.async_copy(
                    table_hbm.at[idx_v.at[j]],
                    rows_v.at[pl.ds(j * CHUNK, CHUNK)],
                    sem,
                )
            )
        for c in copies:
            c.wait()
        pltpu.sync_copy(rows_v, out_hbm.at[pl.ds(base, ROWS_PER_W)])

    return _sc_gather


def _tc_body(center_ref, w_ref, out_ref):
    j = pl.program_id(0)
    sign = jnp.where(j < P, 1.0, -1.0)
    acc = lax.dot_general(
        center_ref[...], w_ref[...],
        (((1,), (1,)), ((), ())),
        preferred_element_type=jnp.float32,
    )
    out_ref[...] = jax.nn.sigmoid(sign * acc)


def _tc_cross(gathered):
    return pl.pallas_call(
        _tc_body,
        grid=(NPOS,),
        in_specs=[
            pl.BlockSpec((B, DIM), lambda j: (NPOS, 0)),
            pl.BlockSpec((B, DIM), lambda j: (j, 0)),
        ],
        out_specs=pl.BlockSpec((B, B), lambda j: (0, j)),
        out_shape=jax.ShapeDtypeStruct((B, NPOS * B), jnp.float32),
    )(gathered, gathered)


def kernel(x_center, x_positive, x_negative, emb_table):
    idx_all = jnp.concatenate([
        x_positive.T.reshape(-1),
        x_negative.T.reshape(-1),
        x_center,
    ]).astype(jnp.int32).reshape(TOT // CHUNK, CHUNK)
    gathered = _make_sc_gather()(idx_all, emb_table)
    return _tc_cross(gathered)
